# Initial kernel scaffold; baseline (speedup 1.0000x reference)
import jax
import jax.numpy as jnp
from jax.experimental import pallas as pl


def kernel(unaries, rgb, superpixel_cliques, spatial_ker_weights, bilateral_ker_weights, superpixel_low_weights, superpixel_high_weight, compatibility_matrix):
    raise NotImplementedError("write your pallas kernel here")


# speedup vs baseline: 5.0306x; 5.0306x over previous
import math

import jax
import jax.numpy as jnp
from jax.experimental import pallas as pl
from jax.experimental.pallas import tpu as pltpu

_C = 6
_H = 512
_W = 512
_NSP = 500
_NSPP = 512
_NITER = 5
_RADIUS = 2
_TA = 160.0
_TB = 3.0
_TG = 3.0

_BH = 128
_NBLK = _H // _BH

_DELTAS = list(range(-_RADIUS, _RADIUS + 1))
_SG = {(dy, dx): math.exp(-float(dy * dy + dx * dx) / (2.0 * _TG * _TG))
       for dy in _DELTAS for dx in _DELTAS}
_SA = {(dy, dx): math.exp(-float(dy * dy + dx * dx) / (2.0 * _TA * _TA))
       for dy in _DELTAS for dx in _DELTAS}
_SN = sum(_SG.values())
_NBI = -1.0 / (2.0 * _TB * _TB)


def _ext(get_rows, s, e):
    top, bot = s - _RADIUS, e + _RADIUS
    parts = []
    if top < 0:
        parts.append(get_rows(_H + top, _H))
        top = 0
    parts.append(get_rows(top, min(bot, _H)))
    if bot > _H:
        parts.append(get_rows(0, bot - _H))
    return jnp.concatenate(parts, axis=0) if len(parts) > 1 else parts[0]


def _crf_body(un_ref, rgb_ref, ids_ref, skw_ref, bkw_ref, low_ref,
              high_ref, comp_ref, out_ref, *scratch):
    f32 = jnp.float32
    sm_refs = scratch[0:_C]
    logq_refs = scratch[_C:2 * _C]
    prod_refs = scratch[2 * _C:3 * _C]
    ibn_ref = scratch[3 * _C]

    skw = [[skw_ref[i, j] for j in range(_C)] for i in range(_C)]
    bkw = [[bkw_ref[i, j] for j in range(_C)] for i in range(_C)]
    comp = [[comp_ref[i, j] for j in range(_C)] for i in range(_C)]
    A = [[sum(comp[i][k] * skw[k][j] for k in range(_C)) for j in range(_C)]
         for i in range(_C)]
    Bm = [[sum(comp[i][k] * bkw[k][j] for k in range(_C)) for j in range(_C)]
          for i in range(_C)]
    low = [low_ref[c] for c in range(_C)]
    high = high_ref[0]
    inv_sn = 1.0 / _SN

    for b in range(_NBLK):
        s, e = b * _BH, (b + 1) * _BH
        rgbe = [_ext(lambda a, z, k=k: rgb_ref[k, a:z, :], s, e)
                for k in range(3)]
        rgbc = [m[_RADIUS:_RADIUS + _BH, :] for m in rgbe]
        bn = None
        for dx in _DELTAS:
            rgbx = [jnp.roll(m, dx, axis=1) if dx else m for m in rgbe]
            for dy in _DELTAS:
                rr = [m[_RADIUS - dy:_RADIUS - dy + _BH, :] for m in rgbx]
                csq = ((rgbc[0] - rr[0]) ** 2 + (rgbc[1] - rr[1]) ** 2
                       + (rgbc[2] - rr[2]) ** 2)
                wb = _SA[(dy, dx)] * jnp.exp(_NBI * csq)
                bn = wb if bn is None else bn + wb
        ibn_ref[s:e, :] = 1.0 / bn

    iota_sc = jax.lax.broadcasted_iota(jnp.int32, (_NSPP, _W), 0)

    for c in range(_C):
        out_ref[c] = un_ref[c]

    for _ in range(_NITER):
        for b in range(_NBLK):
            s, e = b * _BH, (b + 1) * _BH
            qb = [out_ref[c, s:e, :] for c in range(_C)]
            m = qb[0]
            for c in range(1, _C):
                m = jnp.maximum(m, qb[c])
            ex = [jnp.exp(qb[c] - m) for c in range(_C)]
            tot = ex[0]
            for c in range(1, _C):
                tot = tot + ex[c]
            inv_tot = 1.0 / tot
            for c in range(_C):
                smv = ex[c] * inv_tot
                sm_refs[c][s:e, :] = smv
                logq_refs[c][s:e, :] = jnp.log(smv + 1e-5)

        def b_body(r, bt):
            row_ids = ids_ref[pl.ds(r, 1), :]
            oh = (iota_sc == row_ids).astype(f32)
            lq_r = jnp.concatenate(
                [logq_refs[c][pl.ds(r, 1), :] for c in range(_C)],
                axis=0)
            return bt + jax.lax.dot_general(
                lq_r, oh, (((1,), (1,)), ((), ())),
                preferred_element_type=f32)

        bt = jax.lax.fori_loop(0, _H, b_body, jnp.zeros((_C, _NSPP), f32))

        def g_body(r, carry):
            row_ids = ids_ref[pl.ds(r, 1), :]
            oh = (iota_sc == row_ids).astype(f32)
            pr = jnp.dot(bt, oh, preferred_element_type=f32)
            for c in range(_C):
                prod_refs[c][pl.ds(r, 1), :] = pr[c:c + 1, :]
            return carry

        jax.lax.fori_loop(0, _H, g_body, 0)

        for b in range(_NBLK):
            s, e = b * _BH, (b + 1) * _BH
            sme = [_ext(lambda a, z, c=c: sm_refs[c][a:z, :], s, e)
                   for c in range(_C)]
            rgbe = [_ext(lambda a, z, k=k: rgb_ref[k, a:z, :], s, e)
                    for k in range(3)]
            rgbc = [m[_RADIUS:_RADIUS + _BH, :] for m in rgbe]
            sp_acc = [None] * _C
            bi_acc = [None] * _C
            for dx in _DELTAS:
                smx = [jnp.roll(m, dx, axis=1) if dx else m for m in sme]
                rgbx = [jnp.roll(m, dx, axis=1) if dx else m for m in rgbe]
                for dy in _DELTAS:
                    rr = [m[_RADIUS - dy:_RADIUS - dy + _BH, :] for m in rgbx]
                    csq = ((rgbc[0] - rr[0]) ** 2 + (rgbc[1] - rr[1]) ** 2
                           + (rgbc[2] - rr[2]) ** 2)
                    wb = _SA[(dy, dx)] * jnp.exp(_NBI * csq)
                    sg = _SG[(dy, dx)]
                    for c in range(_C):
                        r_ = smx[c][_RADIUS - dy:_RADIUS - dy + _BH, :]
                        t0 = sg * r_
                        t1 = wb * r_
                        sp_acc[c] = t0 if sp_acc[c] is None else sp_acc[c] + t0
                        bi_acc[c] = t1 if bi_acc[c] is None else bi_acc[c] + t1
            ibn_b = ibn_ref[s:e, :]
            sp_out = [sp_acc[c] * inv_sn for c in range(_C)]
            bi_out = [bi_acc[c] * ibn_b for c in range(_C)]
            for i in range(_C):
                ft = jnp.exp(prod_refs[i][s:e, :] - logq_refs[i][s:e, :])
                upd = low[i] * ft + high * (1.0 - ft)
                pw = None
                for j in range(_C):
                    t = A[i][j] * sp_out[j] + Bm[i][j] * bi_out[j]
                    pw = t if pw is None else pw + t
                out_ref[i, s:e, :] = un_ref[i, s:e, :] - pw - upd


def kernel(unaries, rgb, superpixel_cliques, spatial_ker_weights,
           bilateral_ker_weights, superpixel_low_weights,
           superpixel_high_weight, compatibility_matrix):
    un = jnp.transpose(unaries[0], (2, 0, 1))
    rgbt = jnp.transpose(rgb[0], (2, 0, 1))
    ids = jnp.transpose(superpixel_cliques[0])

    res = pl.pallas_call(
        _crf_body,
        out_shape=jax.ShapeDtypeStruct((_C, _H, _W), jnp.float32),
        in_specs=[
            pl.BlockSpec(memory_space=pltpu.VMEM),
            pl.BlockSpec(memory_space=pltpu.VMEM),
            pl.BlockSpec(memory_space=pltpu.VMEM),
            pl.BlockSpec(memory_space=pltpu.SMEM),
            pl.BlockSpec(memory_space=pltpu.SMEM),
            pl.BlockSpec(memory_space=pltpu.SMEM),
            pl.BlockSpec(memory_space=pltpu.SMEM),
            pl.BlockSpec(memory_space=pltpu.SMEM),
        ],
        out_specs=pl.BlockSpec(memory_space=pltpu.VMEM),
        scratch_shapes=(
            [pltpu.VMEM((_H, _W), jnp.float32) for _ in range(3 * _C)]
            + [pltpu.VMEM((_H, _W), jnp.float32)]),
        compiler_params=pltpu.CompilerParams(
            vmem_limit_bytes=64 * 1024 * 1024),
    )(un, rgbt, ids, spatial_ker_weights, bilateral_ker_weights,
      superpixel_low_weights, superpixel_high_weight, compatibility_matrix)
    return jnp.transpose(res, (1, 2, 0))[None]


# speedup vs baseline: 5.8262x; 1.1582x over previous
import math

import jax
import jax.numpy as jnp
from jax.experimental import pallas as pl
from jax.experimental.pallas import tpu as pltpu

_C = 6
_H = 512
_W = 512
_NSP = 500
_NSPP = 512
_NITER = 5
_RADIUS = 2
_TA = 160.0
_TB = 3.0
_TG = 3.0

_BH = 128
_NBLK = _H // _BH

_DELTAS = list(range(-_RADIUS, _RADIUS + 1))
_SG = {(dy, dx): math.exp(-float(dy * dy + dx * dx) / (2.0 * _TG * _TG))
       for dy in _DELTAS for dx in _DELTAS}
_SA = {(dy, dx): math.exp(-float(dy * dy + dx * dx) / (2.0 * _TA * _TA))
       for dy in _DELTAS for dx in _DELTAS}
_SN = sum(_SG.values())
_NBI = -1.0 / (2.0 * _TB * _TB)


def _ext(get_rows, s, e):
    top, bot = s - _RADIUS, e + _RADIUS
    parts = []
    if top < 0:
        parts.append(get_rows(_H + top, _H))
        top = 0
    parts.append(get_rows(top, min(bot, _H)))
    if bot > _H:
        parts.append(get_rows(0, bot - _H))
    return jnp.concatenate(parts, axis=0) if len(parts) > 1 else parts[0]


def _crf_body(un_ref, rgb_ref, ids_ref, skw_ref, bkw_ref, low_ref,
              high_ref, comp_ref, out_ref, *scratch):
    f32 = jnp.float32
    sm_refs = scratch[0:_C]
    logq_refs = scratch[_C:2 * _C]
    prod_refs = scratch[2 * _C:3 * _C]
    ibn_ref = scratch[3 * _C]

    skw = [[skw_ref[i, j] for j in range(_C)] for i in range(_C)]
    bkw = [[bkw_ref[i, j] for j in range(_C)] for i in range(_C)]
    comp = [[comp_ref[i, j] for j in range(_C)] for i in range(_C)]
    A = [[sum(comp[i][k] * skw[k][j] for k in range(_C)) for j in range(_C)]
         for i in range(_C)]
    Bm = [[sum(comp[i][k] * bkw[k][j] for k in range(_C)) for j in range(_C)]
          for i in range(_C)]
    low = [low_ref[c] for c in range(_C)]
    high = high_ref[0]
    inv_sn = 1.0 / _SN

    for b in range(_NBLK):
        s, e = b * _BH, (b + 1) * _BH
        rgbe = [_ext(lambda a, z, k=k: rgb_ref[k, a:z, :], s, e)
                for k in range(3)]
        rgbc = [m[_RADIUS:_RADIUS + _BH, :] for m in rgbe]
        bn = None
        for dx in _DELTAS:
            rgbx = [jnp.roll(m, dx, axis=1) if dx else m for m in rgbe]
            for dy in _DELTAS:
                rr = [m[_RADIUS - dy:_RADIUS - dy + _BH, :] for m in rgbx]
                csq = ((rgbc[0] - rr[0]) ** 2 + (rgbc[1] - rr[1]) ** 2
                       + (rgbc[2] - rr[2]) ** 2)
                wb = _SA[(dy, dx)] * jnp.exp(_NBI * csq)
                bn = wb if bn is None else bn + wb
        ibn_ref[s:e, :] = 1.0 / bn

    iota16 = jax.lax.broadcasted_iota(jnp.int32, (16, _W), 0)
    iota32 = jax.lax.broadcasted_iota(jnp.int32, (32, _W), 0)

    for c in range(_C):
        out_ref[c] = un_ref[c]

    for _ in range(_NITER):
        for b in range(_NBLK):
            s, e = b * _BH, (b + 1) * _BH
            qb = [out_ref[c, s:e, :] for c in range(_C)]
            m = qb[0]
            for c in range(1, _C):
                m = jnp.maximum(m, qb[c])
            ex = [jnp.exp(qb[c] - m) for c in range(_C)]
            tot = ex[0]
            for c in range(1, _C):
                tot = tot + ex[c]
            inv_tot = 1.0 / tot
            for c in range(_C):
                smv = ex[c] * inv_tot
                sm_refs[c][s:e, :] = smv
                logq_refs[c][s:e, :] = jnp.log(smv + 1e-5)

        def b_body(r, bt):
            row_ids = ids_ref[pl.ds(r, 1), :]
            hi = jnp.right_shift(row_ids, 5)
            lo = jnp.bitwise_and(row_ids, 31)
            oh_hi = (iota16 == hi).astype(f32)
            oh_lo = (iota32 == lo).astype(f32)
            lq_r = jnp.concatenate(
                [logq_refs[c][pl.ds(r, 1), :] for c in range(_C)],
                axis=0)
            z = jnp.concatenate(
                [lq_r * oh_hi[h:h + 1, :] for h in range(16)],
                axis=0)
            return bt + jax.lax.dot_general(
                z, oh_lo, (((1,), (1,)), ((), ())),
                preferred_element_type=f32)

        bt = jax.lax.fori_loop(
            0, _H, b_body, jnp.zeros((16 * _C, 32), f32))

        def g_body(r, carry):
            row_ids = ids_ref[pl.ds(r, 1), :]
            hi = jnp.right_shift(row_ids, 5)
            lo = jnp.bitwise_and(row_ids, 31)
            oh_hi = (iota16 == hi).astype(f32)
            oh_lo = (iota32 == lo).astype(f32)
            u = jnp.dot(bt, oh_lo, preferred_element_type=f32)
            pr = None
            for h in range(16):
                t = u[h * _C:(h + 1) * _C, :] * oh_hi[h:h + 1, :]
                pr = t if pr is None else pr + t
            for c in range(_C):
                prod_refs[c][pl.ds(r, 1), :] = pr[c:c + 1, :]
            return carry

        jax.lax.fori_loop(0, _H, g_body, 0)

        for b in range(_NBLK):
            s, e = b * _BH, (b + 1) * _BH
            sme = [_ext(lambda a, z, c=c: sm_refs[c][a:z, :], s, e)
                   for c in range(_C)]
            rgbe = [_ext(lambda a, z, k=k: rgb_ref[k, a:z, :], s, e)
                    for k in range(3)]
            rgbc = [m[_RADIUS:_RADIUS + _BH, :] for m in rgbe]
            sp_acc = [None] * _C
            bi_acc = [None] * _C
            for dx in _DELTAS:
                smx = [jnp.roll(m, dx, axis=1) if dx else m for m in sme]
                rgbx = [jnp.roll(m, dx, axis=1) if dx else m for m in rgbe]
                for dy in _DELTAS:
                    rr = [m[_RADIUS - dy:_RADIUS - dy + _BH, :] for m in rgbx]
                    csq = ((rgbc[0] - rr[0]) ** 2 + (rgbc[1] - rr[1]) ** 2
                           + (rgbc[2] - rr[2]) ** 2)
                    wb = _SA[(dy, dx)] * jnp.exp(_NBI * csq)
                    sg = _SG[(dy, dx)]
                    for c in range(_C):
                        r_ = smx[c][_RADIUS - dy:_RADIUS - dy + _BH, :]
                        t0 = sg * r_
                        t1 = wb * r_
                        sp_acc[c] = t0 if sp_acc[c] is None else sp_acc[c] + t0
                        bi_acc[c] = t1 if bi_acc[c] is None else bi_acc[c] + t1
            ibn_b = ibn_ref[s:e, :]
            sp_out = [sp_acc[c] * inv_sn for c in range(_C)]
            bi_out = [bi_acc[c] * ibn_b for c in range(_C)]
            for i in range(_C):
                ft = jnp.exp(prod_refs[i][s:e, :] - logq_refs[i][s:e, :])
                upd = low[i] * ft + high * (1.0 - ft)
                pw = None
                for j in range(_C):
                    t = A[i][j] * sp_out[j] + Bm[i][j] * bi_out[j]
                    pw = t if pw is None else pw + t
                out_ref[i, s:e, :] = un_ref[i, s:e, :] - pw - upd


def kernel(unaries, rgb, superpixel_cliques, spatial_ker_weights,
           bilateral_ker_weights, superpixel_low_weights,
           superpixel_high_weight, compatibility_matrix):
    un = jnp.transpose(unaries[0], (2, 0, 1))
    rgbt = jnp.transpose(rgb[0], (2, 0, 1))
    ids = jnp.transpose(superpixel_cliques[0])

    res = pl.pallas_call(
        _crf_body,
        out_shape=jax.ShapeDtypeStruct((_C, _H, _W), jnp.float32),
        in_specs=[
            pl.BlockSpec(memory_space=pltpu.VMEM),
            pl.BlockSpec(memory_space=pltpu.VMEM),
            pl.BlockSpec(memory_space=pltpu.VMEM),
            pl.BlockSpec(memory_space=pltpu.SMEM),
            pl.BlockSpec(memory_space=pltpu.SMEM),
            pl.BlockSpec(memory_space=pltpu.SMEM),
            pl.BlockSpec(memory_space=pltpu.SMEM),
            pl.BlockSpec(memory_space=pltpu.SMEM),
        ],
        out_specs=pl.BlockSpec(memory_space=pltpu.VMEM),
        scratch_shapes=(
            [pltpu.VMEM((_H, _W), jnp.float32) for _ in range(3 * _C)]
            + [pltpu.VMEM((_H, _W), jnp.float32)]),
        compiler_params=pltpu.CompilerParams(
            vmem_limit_bytes=64 * 1024 * 1024),
    )(un, rgbt, ids, spatial_ker_weights, bilateral_ker_weights,
      superpixel_low_weights, superpixel_high_weight, compatibility_matrix)
    return jnp.transpose(res, (1, 2, 0))[None]


# speedup vs baseline: 6.2651x; 1.0753x over previous
import math

import jax
import jax.numpy as jnp
from jax.experimental import pallas as pl
from jax.experimental.pallas import tpu as pltpu

_C = 6
_H = 512
_W = 512
_NSP = 500
_NSPP = 512
_NITER = 5
_RADIUS = 2
_TA = 160.0
_TB = 3.0
_TG = 3.0

_BH = 128
_NBLK = _H // _BH

_DELTAS = list(range(-_RADIUS, _RADIUS + 1))
_SG = {(dy, dx): math.exp(-float(dy * dy + dx * dx) / (2.0 * _TG * _TG))
       for dy in _DELTAS for dx in _DELTAS}
_SA = {(dy, dx): math.exp(-float(dy * dy + dx * dx) / (2.0 * _TA * _TA))
       for dy in _DELTAS for dx in _DELTAS}
_SN = sum(_SG.values())
_NBI = -1.0 / (2.0 * _TB * _TB)


def _ext(get_rows, s, e):
    top, bot = s - _RADIUS, e + _RADIUS
    parts = []
    if top < 0:
        parts.append(get_rows(_H + top, _H))
        top = 0
    parts.append(get_rows(top, min(bot, _H)))
    if bot > _H:
        parts.append(get_rows(0, bot - _H))
    return jnp.concatenate(parts, axis=0) if len(parts) > 1 else parts[0]


def _crf_body(un_ref, rgb_ref, ids_ref, skw_ref, bkw_ref, low_ref,
              high_ref, comp_ref, out_ref, *scratch):
    f32 = jnp.float32
    sm_refs = scratch[0:_C]
    logq_refs = scratch[_C:2 * _C]
    prod_refs = scratch[2 * _C:3 * _C]
    ibn_ref = scratch[3 * _C]
    wb_refs = scratch[3 * _C + 1:]

    skw = [[skw_ref[i, j] for j in range(_C)] for i in range(_C)]
    bkw = [[bkw_ref[i, j] for j in range(_C)] for i in range(_C)]
    comp = [[comp_ref[i, j] for j in range(_C)] for i in range(_C)]
    A = [[sum(comp[i][k] * skw[k][j] for k in range(_C)) for j in range(_C)]
         for i in range(_C)]
    Bm = [[sum(comp[i][k] * bkw[k][j] for k in range(_C)) for j in range(_C)]
          for i in range(_C)]
    low = [low_ref[c] for c in range(_C)]
    high = high_ref[0]
    inv_sn = 1.0 / _SN

    for b in range(_NBLK):
        s, e = b * _BH, (b + 1) * _BH
        rgbe = [_ext(lambda a, z, k=k: rgb_ref[k, a:z, :], s, e)
                for k in range(3)]
        rgbc = [m[_RADIUS:_RADIUS + _BH, :] for m in rgbe]
        bn = None
        for dx in _DELTAS:
            rgbx = [jnp.roll(m, dx, axis=1) if dx else m for m in rgbe]
            for dy in _DELTAS:
                rr = [m[_RADIUS - dy:_RADIUS - dy + _BH, :] for m in rgbx]
                csq = ((rgbc[0] - rr[0]) ** 2 + (rgbc[1] - rr[1]) ** 2
                       + (rgbc[2] - rr[2]) ** 2)
                wb = _SA[(dy, dx)] * jnp.exp(_NBI * csq)
                wb_refs[(dy + _RADIUS) * 5 + dx + _RADIUS][s:e, :] = (
                    wb.astype(jnp.bfloat16))
                bn = wb if bn is None else bn + wb
        ibn_ref[s:e, :] = 1.0 / bn

    iota16 = jax.lax.broadcasted_iota(jnp.int32, (16, _W), 0)
    iota32 = jax.lax.broadcasted_iota(jnp.int32, (32, _W), 0)

    for c in range(_C):
        out_ref[c] = un_ref[c]

    for _ in range(_NITER):
        for b in range(_NBLK):
            s, e = b * _BH, (b + 1) * _BH
            qb = [out_ref[c, s:e, :] for c in range(_C)]
            m = qb[0]
            for c in range(1, _C):
                m = jnp.maximum(m, qb[c])
            ex = [jnp.exp(qb[c] - m) for c in range(_C)]
            tot = ex[0]
            for c in range(1, _C):
                tot = tot + ex[c]
            inv_tot = 1.0 / tot
            for c in range(_C):
                smv = ex[c] * inv_tot
                sm_refs[c][s:e, :] = smv
                logq_refs[c][s:e, :] = jnp.log(smv + 1e-5)

        def b_body(r, bt):
            row_ids = ids_ref[pl.ds(r, 1), :]
            hi = jnp.right_shift(row_ids, 5)
            lo = jnp.bitwise_and(row_ids, 31)
            oh_hi = (iota16 == hi).astype(f32)
            oh_lo = (iota32 == lo).astype(f32)
            lq_r = jnp.concatenate(
                [logq_refs[c][pl.ds(r, 1), :] for c in range(_C)],
                axis=0)
            z = jnp.concatenate(
                [lq_r * oh_hi[h:h + 1, :] for h in range(16)],
                axis=0)
            return bt + jax.lax.dot_general(
                z, oh_lo, (((1,), (1,)), ((), ())),
                preferred_element_type=f32)

        bt = jax.lax.fori_loop(
            0, _H, b_body, jnp.zeros((16 * _C, 32), f32))

        def g_body(r, carry):
            row_ids = ids_ref[pl.ds(r, 1), :]
            hi = jnp.right_shift(row_ids, 5)
            lo = jnp.bitwise_and(row_ids, 31)
            oh_hi = (iota16 == hi).astype(f32)
            oh_lo = (iota32 == lo).astype(f32)
            u = jnp.dot(bt, oh_lo, preferred_element_type=f32)
            pr = None
            for h in range(16):
                t = u[h * _C:(h + 1) * _C, :] * oh_hi[h:h + 1, :]
                pr = t if pr is None else pr + t
            for c in range(_C):
                prod_refs[c][pl.ds(r, 1), :] = pr[c:c + 1, :]
            return carry

        jax.lax.fori_loop(0, _H, g_body, 0)

        for b in range(_NBLK):
            s, e = b * _BH, (b + 1) * _BH
            sme = [_ext(lambda a, z, c=c: sm_refs[c][a:z, :], s, e)
                   for c in range(_C)]
            sp_acc = [None] * _C
            bi_acc = [None] * _C
            for dx in _DELTAS:
                smx = [jnp.roll(m, dx, axis=1) if dx else m for m in sme]
                for dy in _DELTAS:
                    wb = wb_refs[(dy + _RADIUS) * 5 + dx + _RADIUS][
                        s:e, :].astype(f32)
                    sg = _SG[(dy, dx)]
                    for c in range(_C):
                        r_ = smx[c][_RADIUS - dy:_RADIUS - dy + _BH, :]
                        t0 = sg * r_
                        t1 = wb * r_
                        sp_acc[c] = t0 if sp_acc[c] is None else sp_acc[c] + t0
                        bi_acc[c] = t1 if bi_acc[c] is None else bi_acc[c] + t1
            ibn_b = ibn_ref[s:e, :]
            sp_out = [sp_acc[c] * inv_sn for c in range(_C)]
            bi_out = [bi_acc[c] * ibn_b for c in range(_C)]
            for i in range(_C):
                ft = jnp.exp(prod_refs[i][s:e, :] - logq_refs[i][s:e, :])
                upd = low[i] * ft + high * (1.0 - ft)
                pw = None
                for j in range(_C):
                    t = A[i][j] * sp_out[j] + Bm[i][j] * bi_out[j]
                    pw = t if pw is None else pw + t
                out_ref[i, s:e, :] = un_ref[i, s:e, :] - pw - upd


def kernel(unaries, rgb, superpixel_cliques, spatial_ker_weights,
           bilateral_ker_weights, superpixel_low_weights,
           superpixel_high_weight, compatibility_matrix):
    un = jnp.transpose(unaries[0], (2, 0, 1))
    rgbt = jnp.transpose(rgb[0], (2, 0, 1))
    ids = jnp.transpose(superpixel_cliques[0])

    res = pl.pallas_call(
        _crf_body,
        out_shape=jax.ShapeDtypeStruct((_C, _H, _W), jnp.float32),
        in_specs=[
            pl.BlockSpec(memory_space=pltpu.VMEM),
            pl.BlockSpec(memory_space=pltpu.VMEM),
            pl.BlockSpec(memory_space=pltpu.VMEM),
            pl.BlockSpec(memory_space=pltpu.SMEM),
            pl.BlockSpec(memory_space=pltpu.SMEM),
            pl.BlockSpec(memory_space=pltpu.SMEM),
            pl.BlockSpec(memory_space=pltpu.SMEM),
            pl.BlockSpec(memory_space=pltpu.SMEM),
        ],
        out_specs=pl.BlockSpec(memory_space=pltpu.VMEM),
        scratch_shapes=(
            [pltpu.VMEM((_H, _W), jnp.float32) for _ in range(3 * _C)]
            + [pltpu.VMEM((_H, _W), jnp.float32)]
            + [pltpu.VMEM((_H, _W), jnp.bfloat16) for _ in range(25)]),
        compiler_params=pltpu.CompilerParams(
            vmem_limit_bytes=64 * 1024 * 1024),
    )(un, rgbt, ids, spatial_ker_weights, bilateral_ker_weights,
      superpixel_low_weights, superpixel_high_weight, compatibility_matrix)
    return jnp.transpose(res, (1, 2, 0))[None]


# speedup vs baseline: 10.3664x; 1.6546x over previous
import math

import jax
import jax.numpy as jnp
from jax.experimental import pallas as pl
from jax.experimental.pallas import tpu as pltpu

_C = 6
_H = 512
_W = 512
_NSP = 500
_NSPP = 512
_NITER = 5
_RADIUS = 2
_TA = 160.0
_TB = 3.0
_TG = 3.0

_BH = 128
_NBLK = _H // _BH
_RB = 4

_DELTAS = list(range(-_RADIUS, _RADIUS + 1))
_SG = {(dy, dx): math.exp(-float(dy * dy + dx * dx) / (2.0 * _TG * _TG))
       for dy in _DELTAS for dx in _DELTAS}
_SA = {(dy, dx): math.exp(-float(dy * dy + dx * dx) / (2.0 * _TA * _TA))
       for dy in _DELTAS for dx in _DELTAS}
_SN = sum(_SG.values())
_NBI = -1.0 / (2.0 * _TB * _TB)


def _ext(get_rows, s, e):
    top, bot = s - _RADIUS, e + _RADIUS
    parts = []
    if top < 0:
        parts.append(get_rows(_H + top, _H))
        top = 0
    parts.append(get_rows(top, min(bot, _H)))
    if bot > _H:
        parts.append(get_rows(0, bot - _H))
    return jnp.concatenate(parts, axis=0) if len(parts) > 1 else parts[0]


def _crf_body(un_ref, rgb_ref, ids_ref, skw_ref, bkw_ref, low_ref,
              high_ref, comp_ref, out_ref, *scratch):
    f32 = jnp.float32
    sm_refs = scratch[0:_C]
    logq_refs = scratch[_C:2 * _C]
    prod_refs = scratch[2 * _C:3 * _C]
    ibn_ref = scratch[3 * _C]
    wb_refs = scratch[3 * _C + 1:]

    skw = [[skw_ref[i, j] for j in range(_C)] for i in range(_C)]
    bkw = [[bkw_ref[i, j] for j in range(_C)] for i in range(_C)]
    comp = [[comp_ref[i, j] for j in range(_C)] for i in range(_C)]
    A = [[sum(comp[i][k] * skw[k][j] for k in range(_C)) for j in range(_C)]
         for i in range(_C)]
    Bm = [[sum(comp[i][k] * bkw[k][j] for k in range(_C)) for j in range(_C)]
          for i in range(_C)]
    low = [low_ref[c] for c in range(_C)]
    high = high_ref[0]
    inv_sn = 1.0 / _SN

    for b in range(_NBLK):
        s, e = b * _BH, (b + 1) * _BH
        rgbe = [_ext(lambda a, z, k=k: rgb_ref[k, a:z, :], s, e)
                for k in range(3)]
        rgbc = [m[_RADIUS:_RADIUS + _BH, :] for m in rgbe]
        bn = None
        for dx in _DELTAS:
            rgbx = [jnp.roll(m, dx, axis=1) if dx else m for m in rgbe]
            for dy in _DELTAS:
                rr = [m[_RADIUS - dy:_RADIUS - dy + _BH, :] for m in rgbx]
                csq = ((rgbc[0] - rr[0]) ** 2 + (rgbc[1] - rr[1]) ** 2
                       + (rgbc[2] - rr[2]) ** 2)
                wb = _SA[(dy, dx)] * jnp.exp(_NBI * csq)
                wb_refs[(dy + _RADIUS) * 5 + dx + _RADIUS][s:e, :] = (
                    wb.astype(jnp.bfloat16))
                bn = wb if bn is None else bn + wb
        ibn_ref[s:e, :] = 1.0 / bn

    iota16 = jax.lax.broadcasted_iota(jnp.int32, (16, _RB * _W), 0)
    iota32 = jax.lax.broadcasted_iota(jnp.int32, (32, _RB * _W), 0)

    for c in range(_C):
        out_ref[c] = un_ref[c]

    for _ in range(_NITER):
        for b in range(_NBLK):
            s, e = b * _BH, (b + 1) * _BH
            qb = [out_ref[c, s:e, :] for c in range(_C)]
            m = qb[0]
            for c in range(1, _C):
                m = jnp.maximum(m, qb[c])
            ex = [jnp.exp(qb[c] - m) for c in range(_C)]
            tot = ex[0]
            for c in range(1, _C):
                tot = tot + ex[c]
            inv_tot = 1.0 / tot
            for c in range(_C):
                smv = ex[c] * inv_tot
                sm_refs[c][s:e, :] = smv
                logq_refs[c][s:e, :] = jnp.log(smv + 1e-5)

        def b_body(rb, bt):
            r0 = rb * _RB
            row_ids = jnp.concatenate(
                [ids_ref[pl.ds(r0 + j, 1), :] for j in range(_RB)],
                axis=1)
            hi = jnp.right_shift(row_ids, 5)
            lo = jnp.bitwise_and(row_ids, 31)
            oh_hi = (iota16 == hi).astype(f32)
            oh_lo = (iota32 == lo).astype(f32)
            lq_r = jnp.concatenate(
                [jnp.concatenate(
                    [logq_refs[c][pl.ds(r0 + j, 1), :] for j in range(_RB)],
                    axis=1) for c in range(_C)],
                axis=0)
            z = jnp.concatenate(
                [lq_r * oh_hi[h:h + 1, :] for h in range(16)],
                axis=0)
            return bt + jax.lax.dot_general(
                z, oh_lo, (((1,), (1,)), ((), ())),
                preferred_element_type=f32)

        bt = jax.lax.fori_loop(
            0, _H // _RB, b_body, jnp.zeros((16 * _C, 32), f32))

        def g_body(rb, carry):
            r0 = rb * _RB
            row_ids = jnp.concatenate(
                [ids_ref[pl.ds(r0 + j, 1), :] for j in range(_RB)],
                axis=1)
            hi = jnp.right_shift(row_ids, 5)
            lo = jnp.bitwise_and(row_ids, 31)
            oh_hi = (iota16 == hi).astype(f32)
            oh_lo = (iota32 == lo).astype(f32)
            u = jnp.dot(bt, oh_lo, preferred_element_type=f32)
            pr = None
            for h in range(16):
                t = u[h * _C:(h + 1) * _C, :] * oh_hi[h:h + 1, :]
                pr = t if pr is None else pr + t
            for j in range(_RB):
                for c in range(_C):
                    prod_refs[c][pl.ds(r0 + j, 1), :] = (
                        pr[c:c + 1, j * _W:(j + 1) * _W])
            return carry

        jax.lax.fori_loop(0, _H // _RB, g_body, 0)

        for b in range(_NBLK):
            s, e = b * _BH, (b + 1) * _BH
            sme = [_ext(lambda a, z, c=c: sm_refs[c][a:z, :], s, e)
                   for c in range(_C)]
            sp_acc = [None] * _C
            bi_acc = [None] * _C
            for dx in _DELTAS:
                smx = [jnp.roll(m, dx, axis=1) if dx else m for m in sme]
                for dy in _DELTAS:
                    wb = wb_refs[(dy + _RADIUS) * 5 + dx + _RADIUS][
                        s:e, :].astype(f32)
                    sg = _SG[(dy, dx)]
                    for c in range(_C):
                        r_ = smx[c][_RADIUS - dy:_RADIUS - dy + _BH, :]
                        t0 = sg * r_
                        t1 = wb * r_
                        sp_acc[c] = t0 if sp_acc[c] is None else sp_acc[c] + t0
                        bi_acc[c] = t1 if bi_acc[c] is None else bi_acc[c] + t1
            ibn_b = ibn_ref[s:e, :]
            sp_out = [sp_acc[c] * inv_sn for c in range(_C)]
            bi_out = [bi_acc[c] * ibn_b for c in range(_C)]
            for i in range(_C):
                ft = jnp.exp(prod_refs[i][s:e, :] - logq_refs[i][s:e, :])
                upd = low[i] * ft + high * (1.0 - ft)
                pw = None
                for j in range(_C):
                    t = A[i][j] * sp_out[j] + Bm[i][j] * bi_out[j]
                    pw = t if pw is None else pw + t
                out_ref[i, s:e, :] = un_ref[i, s:e, :] - pw - upd


def kernel(unaries, rgb, superpixel_cliques, spatial_ker_weights,
           bilateral_ker_weights, superpixel_low_weights,
           superpixel_high_weight, compatibility_matrix):
    un = jnp.transpose(unaries[0], (2, 0, 1))
    rgbt = jnp.transpose(rgb[0], (2, 0, 1))
    ids = jnp.transpose(superpixel_cliques[0])

    res = pl.pallas_call(
        _crf_body,
        out_shape=jax.ShapeDtypeStruct((_C, _H, _W), jnp.float32),
        in_specs=[
            pl.BlockSpec(memory_space=pltpu.VMEM),
            pl.BlockSpec(memory_space=pltpu.VMEM),
            pl.BlockSpec(memory_space=pltpu.VMEM),
            pl.BlockSpec(memory_space=pltpu.SMEM),
            pl.BlockSpec(memory_space=pltpu.SMEM),
            pl.BlockSpec(memory_space=pltpu.SMEM),
            pl.BlockSpec(memory_space=pltpu.SMEM),
            pl.BlockSpec(memory_space=pltpu.SMEM),
        ],
        out_specs=pl.BlockSpec(memory_space=pltpu.VMEM),
        scratch_shapes=(
            [pltpu.VMEM((_H, _W), jnp.float32) for _ in range(3 * _C)]
            + [pltpu.VMEM((_H, _W), jnp.float32)]
            + [pltpu.VMEM((_H, _W), jnp.bfloat16) for _ in range(25)]),
        compiler_params=pltpu.CompilerParams(
            vmem_limit_bytes=64 * 1024 * 1024),
    )(un, rgbt, ids, spatial_ker_weights, bilateral_ker_weights,
      superpixel_low_weights, superpixel_high_weight, compatibility_matrix)
    return jnp.transpose(res, (1, 2, 0))[None]
